# Initial kernel scaffold; baseline (speedup 1.0000x reference)
import jax
import jax.numpy as jnp
from jax.experimental import pallas as pl


def kernel(inputs, W_s, b_s, W_flr, b_flr, W_out, b_out):
    raise NotImplementedError("write your pallas kernel here")


# speedup vs baseline: 6.8763x; 6.8763x over previous
import functools

import jax
import jax.numpy as jnp
from jax.experimental import pallas as pl

_B, _V, _F_IN = 2, 4096, 64
_K, _S_DIM, _F_LR, _F_OUT = 16, 4, 12, 18
_T = 256


def _gravnet_kernel(inp_ref, ws_ref, wf_ref, bf_ref, wo_ref, bo_ref,
                    out_ref):
    i = pl.program_id(1)
    row0 = i * _T

    hi = jax.lax.Precision.HIGHEST
    x = inp_ref[0]
    xt = inp_ref[0, pl.ds(row0, _T), :]

    s_t = jax.lax.dot_general(ws_ref[...], x, (((0,), (1,)), ((), ())),
                              precision=hi, preferred_element_type=jnp.float32)
    st = jnp.dot(xt, ws_ref[...], precision=hi,
                 preferred_element_type=jnp.float32)

    f_t = jax.lax.dot_general(wf_ref[...], x, (((0,), (1,)), ((), ())),
                              precision=hi, preferred_element_type=jnp.float32)
    f_t = f_t + bf_ref[...]

    d2 = jnp.zeros((_T, _V), jnp.float32)
    for d in range(_S_DIM):
        diff = st[:, d:d + 1] - s_t[d:d + 1, :]
        d2 = d2 + diff * diff

    rows = row0 + jax.lax.broadcasted_iota(jnp.int32, (_T, _V), 0)
    cols = jax.lax.broadcasted_iota(jnp.int32, (_T, _V), 1)
    big = jnp.float32(3e38)
    work = jnp.where(rows == cols, big, d2)

    sel = jnp.zeros((_T, _V), dtype=jnp.bool_)
    for _ in range(_K):
        m = jnp.min(work, axis=1, keepdims=True)
        idx = jnp.min(jnp.where(work == m, cols, _V), axis=1, keepdims=True)
        oh = cols == idx
        sel = jnp.logical_or(sel, oh)
        work = jnp.where(oh, big, work)

    w = jnp.where(sel, jnp.exp(-10.0 * d2), 0.0)

    mean = jax.lax.dot_general(w, f_t, (((1,), (1,)), ((), ())),
                               precision=hi, preferred_element_type=jnp.float32)
    mean = mean * (1.0 / _K)

    neg = jnp.float32(-3e38)
    mx_cols = []
    for f in range(_F_LR):
        v = jnp.where(sel, w * f_t[f:f + 1, :], neg)
        mx_cols.append(jnp.max(v, axis=1, keepdims=True))
    mx = jnp.concatenate(mx_cols, axis=1)

    cat = jnp.concatenate([xt, mean, mx], axis=1)
    o = jnp.tanh(jnp.dot(cat, wo_ref[...], precision=hi,
                         preferred_element_type=jnp.float32) + bo_ref[0])
    out_ref[0] = o


@jax.jit
def kernel(inputs, W_s, b_s, W_flr, b_flr, W_out, b_out):
    grid = (_B, _V // _T)
    return pl.pallas_call(
        _gravnet_kernel,
        grid=grid,
        in_specs=[
            pl.BlockSpec((1, _V, _F_IN), lambda b, i: (b, 0, 0)),
            pl.BlockSpec((_F_IN, _S_DIM), lambda b, i: (0, 0)),
            pl.BlockSpec((_F_IN, _F_LR), lambda b, i: (0, 0)),
            pl.BlockSpec((_F_LR, 1), lambda b, i: (0, 0)),
            pl.BlockSpec((_F_IN + 2 * _F_LR, _F_OUT), lambda b, i: (0, 0)),
            pl.BlockSpec((1, _F_OUT), lambda b, i: (0, 0)),
        ],
        out_specs=pl.BlockSpec((1, _T, _F_OUT), lambda b, i: (b, i, 0)),
        out_shape=jax.ShapeDtypeStruct((_B, _V, _F_OUT), jnp.float32),
    )(inputs, W_s, W_flr, b_flr.reshape(_F_LR, 1),
      W_out, b_out.reshape(1, _F_OUT))
